# baseline (speedup 1.0000x reference)
import functools

import jax
import jax.numpy as jnp
from jax import lax
from jax.experimental import pallas as pl
from jax.experimental.pallas import tpu as pltpu
from jax.experimental.pallas import tpu_sc as plsc

NC, NS = 2, 16
NW = NC * NS
B = 16384
D = 64
L = 16
BPW = B // NW
IDX_ROWS = B // 128 // NW
NSEM = 4


def _sc_dot_body(u_hbm, i_hbm, ut_hbm, it_hbm, out_hbm,
                 uidx, iidx, ubuf, ibuf, ishared, wbuf, outv, *sems):
    wid = lax.axis_index("s") * NC + lax.axis_index("c")
    sid = lax.axis_index("s")
    row0 = wid * IDX_ROWS

    pltpu.sync_copy(u_hbm.at[pl.ds(row0, IDX_ROWS)], uidx)
    pltpu.sync_copy(i_hbm.at[pl.ds(row0, IDX_ROWS)], iidx)

    def fire(t, carry):
        uv = uidx[t // 8, pl.ds((t % 8) * L, L)]
        iv = iidx[t // 8, pl.ds((t % 8) * L, L)]
        for j in range(L):
            p = t * (L // 2) + j // 2
            h = (j % 2) * D
            pltpu.async_copy(ut_hbm.at[uv[j]], ubuf.at[p, pl.ds(h, D)],
                             sems[0])
            pltpu.async_copy(it_hbm.at[iv[j]],
                             ishared.at[sid, p, pl.ds(h, D)], sems[1])
        return carry
    lax.fori_loop(0, BPW // L, fire, 0)

    def drain(t, carry):
        pltpu.make_async_copy(
            ut_hbm.at[0], ubuf.at[0, pl.ds(0, D)], sems[0]).wait()
        pltpu.make_async_copy(
            it_hbm.at[0], ishared.at[0, 0, pl.ds(0, D)], sems[1]).wait()
        return carry
    lax.fori_loop(0, BPW, drain, 0)

    pltpu.sync_copy(ishared.at[sid], ibuf)

    def pair_body(p, carry):
        for h in range(2):
            o = h * D
            w = ubuf[p, pl.ds(o, L)] * ibuf[p, pl.ds(o, L)]
            w += ubuf[p, pl.ds(o + L, L)] * ibuf[p, pl.ds(o + L, L)]
            w += ubuf[p, pl.ds(o + 2 * L, L)] * ibuf[p, pl.ds(o + 2 * L, L)]
            w += ubuf[p, pl.ds(o + 3 * L, L)] * ibuf[p, pl.ds(o + 3 * L, L)]
            wbuf[pl.ds((2 * p + h) * L, L)] = w
        return carry
    lax.fori_loop(0, BPW // 2, pair_body, 0)

    def grp_body(g, carry):
        j0 = g * L
        base_ids = (j0 + lax.iota(jnp.int32, L)) * L
        acc = plsc.load_gather(wbuf, [base_ids])
        for l in range(1, L):
            acc += plsc.load_gather(wbuf, [base_ids + l])
        outv[pl.ds(j0, L)] = acc
        return carry
    lax.fori_loop(0, BPW // L, grp_body, 0)

    pltpu.sync_copy(outv, out_hbm.at[pl.ds(wid * BPW, BPW)])


def kernel(user, item, user_table, item_table):
    user2d = user.reshape(128, 128)
    item2d = item.reshape(128, 128)
    mesh = plsc.VectorSubcoreMesh(core_axis_name="c", subcore_axis_name="s")
    out = pl.kernel(
        _sc_dot_body,
        out_type=jax.ShapeDtypeStruct((B,), jnp.float32),
        mesh=mesh,
        compiler_params=pltpu.CompilerParams(needs_layout_passes=False),
        scratch_types=[
            pltpu.VMEM((IDX_ROWS, 128), jnp.int32),
            pltpu.VMEM((IDX_ROWS, 128), jnp.int32),
            pltpu.VMEM((BPW // 2, 2 * D), jnp.float32),
            pltpu.VMEM((BPW // 2, 2 * D), jnp.float32),
            pltpu.VMEM_SHARED((NS, BPW // 2, 2 * D), jnp.float32),
            pltpu.VMEM((BPW * L,), jnp.float32),
            pltpu.VMEM((BPW,), jnp.float32),
        ] + [pltpu.SemaphoreType.DMA] * NSEM,
    )(user2d, item2d, user_table, item_table)
    return out
